# baseline (speedup 1.0000x reference)
import jax
import jax.numpy as jnp
from jax import lax
from jax.experimental import pallas as pl
from jax.experimental.pallas import tpu as pltpu
from jax.experimental.pallas import tpu_sc as plsc

N = 16384
EMB = 4096
E = 64
K = 8

BT = 512
CHUNK_ROWS = (12288, 4096)

_NC = 2
_NS = 16
_NW = _NC * _NS


def _matmul_t_block(x_ref, w_ref, b_ref, out_ref):
    out_ref[...] = (
        lax.dot_general(
            w_ref[...], x_ref[...],
            (((1,), (1,)), ((), ())),
            preferred_element_type=jnp.float32,
        )
        + b_ref[...]
    )


def _make_sc_route(rows):
    rpw = rows // _NW
    half = rpw // 2
    ngrp = half // 16

    def body(lt_hbm, scores_hbm, idx_hbm, lt_v, sc_v, ix_v):
        cid = lax.axis_index("c")
        sid = lax.axis_index("s")
        wid = sid * _NC + cid
        base = wid * rpw
        pltpu.sync_copy(lt_hbm.at[:, pl.ds(base, rpw)], lt_v)
        lanes = lax.broadcasted_iota(jnp.int32, (16,), 0)
        zero_row = jnp.zeros((16,), jnp.float32)

        for h in range(2):

            def group(g, carry):
                roff = h * half + g * 16
                lroff = g * 16
                neg_inf = jnp.full((16,), -jnp.inf, jnp.float32)
                t = [neg_inf] * K
                ti = [jnp.zeros((16,), jnp.int32)] * K
                for e in range(E):
                    v = lt_v[e, pl.ds(roff, 16)]
                    ei = jnp.full((16,), e, jnp.int32)
                    for j in range(K if e >= K else e + 1):
                        m = v > t[j]
                        nt = jnp.where(m, v, t[j])
                        ni = jnp.where(m, ei, ti[j])
                        v = jnp.where(m, t[j], v)
                        ei = jnp.where(m, ti[j], ei)
                        t[j] = nt
                        ti[j] = ni
                for r in range(16):
                    for c in range(E // 16):
                        sc_v[lroff + r, pl.ds(c * 16, 16)] = zero_row
                lrows = lanes + lroff
                m0 = t[0]
                exps = [jnp.exp(v - m0) for v in t]
                den = exps[0]
                for ex in exps[1:]:
                    den = den + ex
                rden = jnp.float32(1.0) / den
                for j in range(K):
                    plsc.store_scatter(sc_v, [lrows, ti[j]], exps[j] * rden)
                    plsc.store_scatter(
                        ix_v, [lrows, jnp.full((16,), j, jnp.int32)], ti[j]
                    )
                return carry

            lax.fori_loop(0, ngrp, group, 0)
            out_rows = pl.ds(base + h * half, half)
            pltpu.sync_copy(sc_v, scores_hbm.at[out_rows, :])
            pltpu.sync_copy(ix_v, idx_hbm.at[out_rows, :])

    return pl.kernel(
        body,
        out_type=[
            jax.ShapeDtypeStruct((rows, E), jnp.float32),
            jax.ShapeDtypeStruct((rows, K), jnp.int32),
        ],
        mesh=plsc.VectorSubcoreMesh(core_axis_name="c", subcore_axis_name="s"),
        compiler_params=pltpu.CompilerParams(needs_layout_passes=False),
        scratch_types=[
            pltpu.VMEM((E, rpw), jnp.float32),
            pltpu.VMEM((half, E), jnp.float32),
            pltpu.VMEM((half, K), jnp.int32),
        ],
    )


def kernel(x, Wr, br, Wn, bn):
    del Wn, bn
    brow = br.reshape(E, 1)
    parts = []
    row0 = 0
    for rows_c in CHUNK_ROWS:
        c0 = row0 // BT
        logits_t = pl.pallas_call(
            _matmul_t_block,
            grid=(rows_c // BT,),
            in_specs=[
                pl.BlockSpec((BT, EMB), lambda i, c0=c0: (c0 + i, 0)),
                pl.BlockSpec((E, EMB), lambda i: (0, 0)),
                pl.BlockSpec((E, 1), lambda i: (0, 0)),
            ],
            out_specs=pl.BlockSpec((E, BT), lambda i: (0, i)),
            out_shape=jax.ShapeDtypeStruct((E, rows_c), jnp.float32),
        )(x, Wr, brow)
        parts.append(_make_sc_route(rows_c)(logits_t))
        row0 += rows_c
    if len(CHUNK_ROWS) == 1:
        return parts[0]
    return (
        jnp.concatenate([p[0] for p in parts], axis=0),
        jnp.concatenate([p[1] for p in parts], axis=0),
    )


# speedup vs baseline: 1.0520x; 1.0520x over previous
import jax
import jax.numpy as jnp
from jax import lax
from jax.experimental import pallas as pl
from jax.experimental.pallas import tpu as pltpu
from jax.experimental.pallas import tpu_sc as plsc

N = 16384
EMB = 4096
E = 64
K = 8

BT = 512
CHUNKS = 2

_NC = 2
_NS = 16
_NW = _NC * _NS


def _matmul_t_block(x_ref, w_ref, b_ref, out_ref):
    out_ref[...] = (
        lax.dot_general(
            w_ref[...], x_ref[...],
            (((1,), (1,)), ((), ())),
            preferred_element_type=jnp.float32,
        )
        + b_ref[...]
    )


def _make_sc_route(rows):
    rpw = rows // _NW
    half = rpw // 2
    ngrp = half // 16

    def body(lt_hbm, scores_hbm, idx_hbm, lt_v, sc_v, ix_v):
        cid = lax.axis_index("c")
        sid = lax.axis_index("s")
        wid = sid * _NC + cid
        base = wid * rpw
        pltpu.sync_copy(lt_hbm.at[:, pl.ds(base, rpw)], lt_v)
        lanes = lax.broadcasted_iota(jnp.int32, (16,), 0)
        zero_row = jnp.zeros((16,), jnp.float32)

        for h in range(2):

            def group(g, carry):
                roff = h * half + g * 16
                lroff = g * 16
                neg_inf = jnp.full((16,), -jnp.inf, jnp.float32)
                t = [neg_inf] * K
                ti = [jnp.zeros((16,), jnp.int32)] * K
                for e in range(E):
                    v = lt_v[e, pl.ds(roff, 16)]
                    ei = jnp.full((16,), e, jnp.int32)
                    for j in range(K if e >= K else e + 1):
                        m = v > t[j]
                        nt = jnp.where(m, v, t[j])
                        ni = jnp.where(m, ei, ti[j])
                        v = jnp.where(m, t[j], v)
                        ei = jnp.where(m, ti[j], ei)
                        t[j] = nt
                        ti[j] = ni
                for r in range(16):
                    for c in range(E // 16):
                        sc_v[lroff + r, pl.ds(c * 16, 16)] = zero_row
                lrows = lanes + lroff
                m0 = t[0]
                exps = [jnp.exp(v - m0) for v in t]
                den = exps[0]
                for ex in exps[1:]:
                    den = den + ex
                rden = jnp.float32(1.0) / den
                for j in range(K):
                    plsc.store_scatter(sc_v, [lrows, ti[j]], exps[j] * rden)
                    plsc.store_scatter(
                        ix_v, [lrows, jnp.full((16,), j, jnp.int32)], ti[j]
                    )
                return carry

            lax.fori_loop(0, ngrp, group, 0)
            out_rows = pl.ds(base + h * half, half)
            pltpu.sync_copy(sc_v, scores_hbm.at[out_rows, :])
            pltpu.sync_copy(ix_v, idx_hbm.at[out_rows, :])

    return pl.kernel(
        body,
        out_type=[
            jax.ShapeDtypeStruct((rows, E), jnp.float32),
            jax.ShapeDtypeStruct((rows, K), jnp.int32),
        ],
        mesh=plsc.VectorSubcoreMesh(core_axis_name="c", subcore_axis_name="s"),
        compiler_params=pltpu.CompilerParams(needs_layout_passes=False),
        scratch_types=[
            pltpu.VMEM((E, rpw), jnp.float32),
            pltpu.VMEM((half, E), jnp.float32),
            pltpu.VMEM((half, K), jnp.int32),
        ],
    )


def kernel(x, Wr, br, Wn, bn):
    del Wn, bn
    rows_c = N // CHUNKS
    sc_route = _make_sc_route(rows_c)
    brow = br.reshape(E, 1)
    parts = []
    for c in range(CHUNKS):
        c0 = c * (rows_c // BT)
        logits_t = pl.pallas_call(
            _matmul_t_block,
            grid=(rows_c // BT,),
            in_specs=[
                pl.BlockSpec((BT, EMB), lambda i, c0=c0: (c0 + i, 0)),
                pl.BlockSpec((E, EMB), lambda i: (0, 0)),
                pl.BlockSpec((E, 1), lambda i: (0, 0)),
            ],
            out_specs=pl.BlockSpec((E, BT), lambda i: (0, i)),
            out_shape=jax.ShapeDtypeStruct((E, rows_c), jnp.float32),
        )(x, Wr, brow)
        parts.append(sc_route(logits_t))
    if CHUNKS == 1:
        return parts[0]
    return (
        jnp.concatenate([p[0] for p in parts], axis=0),
        jnp.concatenate([p[1] for p in parts], axis=0),
    )
